# Initial kernel scaffold; baseline (speedup 1.0000x reference)
import jax
import jax.numpy as jnp
from jax.experimental import pallas as pl


def kernel(voxel_centers, spatial_features, W1a, W1b, W2a, W2b, Wf, gamma, beta):
    raise NotImplementedError("write your pallas kernel here")


# speedup vs baseline: 7.7860x; 7.7860x over previous
import functools

import jax
import jax.numpy as jnp
from jax import lax
from jax.experimental import pallas as pl
from jax.experimental.pallas import tpu as pltpu
from jax.experimental.pallas import tpu_sc as plsc

_PC_MIN_X = 0.0
_PC_MIN_Y = -40.0
_VOX_X = 0.05
_VOX_Y = 0.05
_STRIDE = 8
_NKP = 2048
_R1, _NS1 = 4.0, 16
_R2, _NS2 = 8.0, 32
_H, _W = 200, 176

_SC_NC, _SC_NS = 2, 16
_SC_NW = _SC_NC * _SC_NS


def _fps_body(x_ref, o_ref):
    X = x_ref[0, 0]
    Y = x_ref[0, 1]
    Z = x_ref[0, 2]
    li = (lax.broadcasted_iota(jnp.int32, (64, 128), 0) * 128
          + lax.broadcasted_iota(jnp.int32, (64, 128), 1))

    def pick(arr, j):
        return jnp.sum(jnp.where(li == j, arr, 0.0))

    x0 = pick(X, 0)
    y0 = pick(Y, 0)
    z0 = pick(Z, 0)
    o_ref[0, 0, 0] = x0
    o_ref[0, 1, 0] = y0
    o_ref[0, 2, 0] = z0

    def body(i, carry):
        mind, lx, ly, lz = carry
        d = (X - lx) ** 2 + (Y - ly) ** 2 + (Z - lz) ** 2
        mind = jnp.minimum(mind, d)
        m = jnp.max(mind)
        cand = jnp.where(mind == m, li, jnp.int32(2 ** 30))
        j = jnp.min(cand)
        nx = pick(X, j)
        ny = pick(Y, j)
        nz = pick(Z, j)
        o_ref[0, 0, i] = nx
        o_ref[0, 1, i] = ny
        o_ref[0, 2, i] = nz
        return (mind, nx, ny, nz)

    mind0 = jnp.full((64, 128), 1e10, jnp.float32)
    lax.fori_loop(1, _NKP, body, (mind0, x0, y0, z0))


def _fps_call(x4):
    B = x4.shape[0]
    return pl.pallas_call(
        _fps_body,
        grid=(B,),
        in_specs=[pl.BlockSpec((1, 3, 64, 128), lambda b: (b, 0, 0, 0))],
        out_specs=pl.BlockSpec((1, 3, _NKP), lambda b: (b, 0, 0),
                               memory_space=pltpu.SMEM),
        out_shape=jax.ShapeDtypeStruct((B, 3, _NKP), jnp.float32),
    )(x4)


def _bev_gather(table, idx):
    n = idx.shape[0]
    per_w = n // _SC_NW
    chunks = per_w // 128
    mesh = plsc.VectorSubcoreMesh(core_axis_name="c", subcore_axis_name="s")

    @functools.partial(
        pl.kernel, mesh=mesh,
        out_type=jax.ShapeDtypeStruct((n, 256), jnp.float32),
        scratch_types=[
            pltpu.VMEM((128,), jnp.int32),
            pltpu.VMEM((128, 256), jnp.float32),
            pltpu.SemaphoreType.DMA,
        ],
    )
    def k(table_hbm, idx_hbm, out_hbm, idx_v, rows_v, sem):
        wid = lax.axis_index("s") * _SC_NC + lax.axis_index("c")
        for c in range(chunks):
            base = wid * per_w + c * 128
            pltpu.sync_copy(idx_hbm.at[pl.ds(base, 128)], idx_v)
            pltpu.async_copy(table_hbm.at[idx_v], rows_v, sem).wait()
            pltpu.sync_copy(rows_v, out_hbm.at[pl.ds(base, 128)])

    return k(table, idx)


_QB = 128


def _bq_body(kp_ref, xt_ref, pts_ref, w1a_ref, w1b_ref, w2a_ref, w2b_ref,
             o1_ref, o2_ref):
    Q = kp_ref[0]
    Xt = xt_ref[0]
    P = pts_ref[0]
    n = Xt.shape[1]
    qn = jnp.sum(Q * Q, axis=1, keepdims=True)
    xn = jnp.sum(Xt * Xt, axis=0, keepdims=True)
    qx = lax.dot_general(Q, Xt, (((1,), (0,)), ((), ())),
                         preferred_element_type=jnp.float32)
    d2 = jnp.maximum(qn + xn - 2.0 * qx, 0.0)
    li = lax.broadcasted_iota(jnp.int32, (_QB, n), 1)
    big_i = jnp.int32(2 ** 30)

    W1a = w1a_ref[...]
    W1b = w1b_ref[...]
    W2a = w2a_ref[...]
    W2b = w2b_ref[...]

    def step(s, carry):
        cur, pooled1, pooled2 = carry
        m = jnp.min(cur, axis=1, keepdims=True)
        cand = jnp.where(cur == m, li, big_i)
        j = jnp.min(cand, axis=1, keepdims=True)
        oh = cand == j
        sel = lax.dot_general(oh.astype(jnp.float32), P,
                              (((1,), (0,)), ((), ())),
                              preferred_element_type=jnp.float32)
        g = sel - Q
        h1 = jnp.maximum(jnp.dot(g, W1a), 0.0)
        h1 = jnp.maximum(jnp.dot(h1, W1b), 0.0)
        h2 = jnp.maximum(jnp.dot(g, W2a), 0.0)
        h2 = jnp.maximum(jnp.dot(h2, W2b), 0.0)
        v1 = (m <= _R1 * _R1) & (s < _NS1)
        pooled1 = jnp.where(v1, jnp.maximum(pooled1, h1), pooled1)
        v2 = m <= _R2 * _R2
        pooled2 = jnp.where(v2, jnp.maximum(pooled2, h2), pooled2)
        cur = jnp.where(oh, 1e10, cur)
        return (cur, pooled1, pooled2)

    _, pooled1, pooled2 = lax.fori_loop(
        0, _NS2, step,
        (d2, jnp.zeros((_QB, _NS1), jnp.float32),
         jnp.zeros((_QB, _NS2), jnp.float32)))
    o1_ref[0] = pooled1
    o2_ref[0] = pooled2


def _bq_call(keypoints, xt, pts, W1a, W1b, W2a, W2b):
    B, N, _ = pts.shape
    grid = (B, _NKP // _QB)
    return pl.pallas_call(
        _bq_body,
        grid=grid,
        in_specs=[
            pl.BlockSpec((1, _QB, 3), lambda b, q: (b, q, 0)),
            pl.BlockSpec((1, 3, N), lambda b, q: (b, 0, 0)),
            pl.BlockSpec((1, N, 3), lambda b, q: (b, 0, 0)),
            pl.BlockSpec((3, _NS1), lambda b, q: (0, 0)),
            pl.BlockSpec((_NS1, _NS1), lambda b, q: (0, 0)),
            pl.BlockSpec((3, _NS2), lambda b, q: (0, 0)),
            pl.BlockSpec((_NS2, _NS2), lambda b, q: (0, 0)),
        ],
        out_specs=[
            pl.BlockSpec((1, _QB, _NS1), lambda b, q: (b, q, 0)),
            pl.BlockSpec((1, _QB, _NS2), lambda b, q: (b, q, 0)),
        ],
        out_shape=[
            jax.ShapeDtypeStruct((B, _NKP, _NS1), jnp.float32),
            jax.ShapeDtypeStruct((B, _NKP, _NS2), jnp.float32),
        ],
    )(keypoints, xt, pts, W1a, W1b, W2a, W2b)


def _fuse_body(rows_ref, w_ref, sa1_ref, sa2_ref, wf_ref, g_ref, b_ref, o_ref):
    bev = rows_ref[0] * w_ref[0]
    for c in range(1, 4):
        bev = bev + rows_ref[c] * w_ref[c]
    feats = jnp.concatenate([bev, sa1_ref[...], sa2_ref[...]], axis=1)
    h = jnp.dot(feats, wf_ref[...], preferred_element_type=jnp.float32)
    mean = jnp.mean(h, axis=0, keepdims=True)
    var = jnp.mean((h - mean) ** 2, axis=0, keepdims=True)
    hn = (h - mean) / jnp.sqrt(var + 1e-5) * g_ref[...] + b_ref[...]
    o_ref[...] = jnp.maximum(hn, 0.0)


def _fuse_call(rows, w4, sa1, sa2, Wf, gamma, beta):
    M = sa1.shape[0]
    return pl.pallas_call(
        _fuse_body,
        out_shape=jax.ShapeDtypeStruct((M, 128), jnp.float32),
    )(rows, w4, sa1, sa2, Wf, gamma.reshape(1, 128), beta.reshape(1, 128))


def kernel(voxel_centers, spatial_features, W1a, W1b, W2a, W2b, Wf, gamma,
           beta):
    B, N, _ = voxel_centers.shape
    K = _NKP
    xt = jnp.transpose(voxel_centers, (0, 2, 1))
    kp_t = _fps_call(xt.reshape(B, 3, N // 128, 128))
    keypoints = jnp.transpose(kp_t, (0, 2, 1))

    xi = (keypoints[:, :, 0] - _PC_MIN_X) / _VOX_X / _STRIDE
    yi = (keypoints[:, :, 1] - _PC_MIN_Y) / _VOX_Y / _STRIDE
    x0 = jnp.floor(xi).astype(jnp.int32)
    x1 = x0 + 1
    y0 = jnp.floor(yi).astype(jnp.int32)
    y1 = y0 + 1
    x0 = jnp.clip(x0, 0, _W - 1)
    x1 = jnp.clip(x1, 0, _W - 1)
    y0 = jnp.clip(y0, 0, _H - 1)
    y1 = jnp.clip(y1, 0, _H - 1)
    x0f = x0.astype(jnp.float32)
    x1f = x1.astype(jnp.float32)
    y0f = y0.astype(jnp.float32)
    y1f = y1.astype(jnp.float32)
    wa = (x1f - xi) * (y1f - yi)
    wb = (x1f - xi) * (yi - y0f)
    wc = (xi - x0f) * (y1f - yi)
    wd = (xi - x0f) * (yi - y0f)
    boff = (jnp.arange(B, dtype=jnp.int32) * (_H * _W))[:, None]
    ia = boff + y0 * _W + x0
    ib = boff + y1 * _W + x0
    ic = boff + y0 * _W + x1
    idd = boff + y1 * _W + x1
    idx = jnp.stack([ia, ib, ic, idd]).reshape(-1)
    table = jnp.transpose(spatial_features, (0, 2, 3, 1)).reshape(
        B * _H * _W, 256)
    rows = _bev_gather(table, idx).reshape(4, B * K, 256)
    w4 = jnp.stack([wa, wb, wc, wd]).reshape(4, B * K, 1)

    pooled1, pooled2 = _bq_call(keypoints, xt, voxel_centers,
                                W1a, W1b, W2a, W2b)
    sa1 = pooled1.reshape(B * K, _NS1)
    sa2 = pooled2.reshape(B * K, _NS2)
    return _fuse_call(rows, w4, sa1, sa2, Wf, gamma, beta)


# speedup vs baseline: 10.0153x; 1.2863x over previous
import functools

import jax
import jax.numpy as jnp
from jax import lax
from jax.experimental import pallas as pl
from jax.experimental.pallas import tpu as pltpu
from jax.experimental.pallas import tpu_sc as plsc

_PC_MIN_X = 0.0
_PC_MIN_Y = -40.0
_VOX_X = 0.05
_VOX_Y = 0.05
_STRIDE = 8
_NKP = 2048
_R1, _NS1 = 4.0, 16
_R2, _NS2 = 8.0, 32
_H, _W = 200, 176

_SC_NC, _SC_NS = 2, 16
_SC_NW = _SC_NC * _SC_NS


def _fps_body(x_ref, xs_ref, o_ref):
    B = x_ref.shape[0]
    li = (lax.broadcasted_iota(jnp.int32, (64, 128), 0) * 128
          + lax.broadcasted_iota(jnp.int32, (64, 128), 1))
    big_i = jnp.int32(2 ** 30)
    P = [(x_ref[b, 0], x_ref[b, 1], x_ref[b, 2]) for b in range(B)]

    for b in range(B):
        o_ref[b, 0, 0] = xs_ref[b, 0, 0]
        o_ref[b, 1, 0] = xs_ref[b, 1, 0]
        o_ref[b, 2, 0] = xs_ref[b, 2, 0]

    def body(i, carry):
        minds, lasts = carry
        new_minds = []
        new_lasts = []
        for b in range(B):
            X, Y, Z = P[b]
            lx, ly, lz = lasts[b]
            d = (X - lx) ** 2 + (Y - ly) ** 2 + (Z - lz) ** 2
            mind = jnp.minimum(minds[b], d)
            m = jnp.max(mind)
            cand = jnp.where(mind == m, li, big_i)
            j = jnp.min(cand)
            nx = xs_ref[b, 0, j]
            ny = xs_ref[b, 1, j]
            nz = xs_ref[b, 2, j]
            o_ref[b, 0, i] = nx
            o_ref[b, 1, i] = ny
            o_ref[b, 2, i] = nz
            new_minds.append(mind)
            new_lasts.append((nx, ny, nz))
        return (tuple(new_minds), tuple(new_lasts))

    mind0 = jnp.full((64, 128), 1e10, jnp.float32)
    lasts0 = tuple((xs_ref[b, 0, 0], xs_ref[b, 1, 0], xs_ref[b, 2, 0])
                   for b in range(B))
    lax.fori_loop(1, _NKP, body, (tuple(mind0 for _ in range(B)), lasts0))


def _fps_call(x4, xt):
    B = x4.shape[0]
    return pl.pallas_call(
        _fps_body,
        in_specs=[
            pl.BlockSpec(memory_space=pltpu.VMEM),
            pl.BlockSpec(memory_space=pltpu.SMEM),
        ],
        out_specs=pl.BlockSpec(memory_space=pltpu.SMEM),
        out_shape=jax.ShapeDtypeStruct((B, 3, _NKP), jnp.float32),
    )(x4, xt)


def _bev_gather(table, idx):
    n = idx.shape[0]
    per_w = n // _SC_NW
    chunks = per_w // 128
    mesh = plsc.VectorSubcoreMesh(core_axis_name="c", subcore_axis_name="s")

    @functools.partial(
        pl.kernel, mesh=mesh,
        out_type=jax.ShapeDtypeStruct((n, 256), jnp.float32),
        scratch_types=[
            pltpu.VMEM((128,), jnp.int32),
            pltpu.VMEM((128, 256), jnp.float32),
            pltpu.SemaphoreType.DMA,
        ],
    )
    def k(table_hbm, idx_hbm, out_hbm, idx_v, rows_v, sem):
        wid = lax.axis_index("s") * _SC_NC + lax.axis_index("c")
        for c in range(chunks):
            base = wid * per_w + c * 128
            pltpu.sync_copy(idx_hbm.at[pl.ds(base, 128)], idx_v)
            pltpu.async_copy(table_hbm.at[idx_v], rows_v, sem).wait()
            pltpu.sync_copy(rows_v, out_hbm.at[pl.ds(base, 128)])

    return k(table, idx)


_QB = 256


def _bq_body(kp_ref, xt_ref, pts_ref, w1a_ref, w1b_ref, w2a_ref, w2b_ref,
             o1_ref, o2_ref):
    Q = kp_ref[0]
    Xt = xt_ref[0]
    P = pts_ref[0]
    n = Xt.shape[1]
    qn = jnp.sum(Q * Q, axis=1, keepdims=True)
    xn = jnp.sum(Xt * Xt, axis=0, keepdims=True)
    qx = lax.dot_general(Q, Xt, (((1,), (0,)), ((), ())),
                         preferred_element_type=jnp.float32)
    d2 = jnp.maximum(qn + xn - 2.0 * qx, 0.0)
    li = lax.broadcasted_iota(jnp.int32, (_QB, n), 1)
    big_i = jnp.int32(2 ** 30)

    W1a = w1a_ref[...]
    W1b = w1b_ref[...]
    W2a = w2a_ref[...]
    W2b = w2b_ref[...]

    def step(s, carry):
        cur, pooled1, pooled2 = carry
        m = jnp.min(cur, axis=1, keepdims=True)
        cand = jnp.where(cur == m, li, big_i)
        j = jnp.min(cand, axis=1, keepdims=True)
        ohf = (cand == j).astype(jnp.float32)
        sel = lax.dot_general(ohf, P,
                              (((1,), (0,)), ((), ())),
                              preferred_element_type=jnp.float32)
        g = sel - Q
        h1 = jnp.maximum(jnp.dot(g, W1a), 0.0)
        h1 = jnp.maximum(jnp.dot(h1, W1b), 0.0)
        h2 = jnp.maximum(jnp.dot(g, W2a), 0.0)
        h2 = jnp.maximum(jnp.dot(h2, W2b), 0.0)
        v1 = (m <= _R1 * _R1) & (s < _NS1)
        pooled1 = jnp.where(v1, jnp.maximum(pooled1, h1), pooled1)
        v2 = m <= _R2 * _R2
        pooled2 = jnp.where(v2, jnp.maximum(pooled2, h2), pooled2)
        cur = cur + ohf * 1e10
        return (cur, pooled1, pooled2)

    _, pooled1, pooled2 = lax.fori_loop(
        0, _NS2, step,
        (d2, jnp.zeros((_QB, _NS1), jnp.float32),
         jnp.zeros((_QB, _NS2), jnp.float32)))
    o1_ref[0] = pooled1
    o2_ref[0] = pooled2


def _bq_call(keypoints, xt, pts, W1a, W1b, W2a, W2b):
    B, N, _ = pts.shape
    grid = (B, _NKP // _QB)
    return pl.pallas_call(
        _bq_body,
        grid=grid,
        in_specs=[
            pl.BlockSpec((1, _QB, 3), lambda b, q: (b, q, 0)),
            pl.BlockSpec((1, 3, N), lambda b, q: (b, 0, 0)),
            pl.BlockSpec((1, N, 3), lambda b, q: (b, 0, 0)),
            pl.BlockSpec((3, _NS1), lambda b, q: (0, 0)),
            pl.BlockSpec((_NS1, _NS1), lambda b, q: (0, 0)),
            pl.BlockSpec((3, _NS2), lambda b, q: (0, 0)),
            pl.BlockSpec((_NS2, _NS2), lambda b, q: (0, 0)),
        ],
        out_specs=[
            pl.BlockSpec((1, _QB, _NS1), lambda b, q: (b, q, 0)),
            pl.BlockSpec((1, _QB, _NS2), lambda b, q: (b, q, 0)),
        ],
        out_shape=[
            jax.ShapeDtypeStruct((B, _NKP, _NS1), jnp.float32),
            jax.ShapeDtypeStruct((B, _NKP, _NS2), jnp.float32),
        ],
    )(keypoints, xt, pts, W1a, W1b, W2a, W2b)


def _fuse_body(rows_ref, w_ref, sa1_ref, sa2_ref, wf_ref, g_ref, b_ref, o_ref):
    bev = rows_ref[0] * w_ref[0]
    for c in range(1, 4):
        bev = bev + rows_ref[c] * w_ref[c]
    feats = jnp.concatenate([bev, sa1_ref[...], sa2_ref[...]], axis=1)
    h = jnp.dot(feats, wf_ref[...], preferred_element_type=jnp.float32)
    mean = jnp.mean(h, axis=0, keepdims=True)
    var = jnp.mean((h - mean) ** 2, axis=0, keepdims=True)
    hn = (h - mean) / jnp.sqrt(var + 1e-5) * g_ref[...] + b_ref[...]
    o_ref[...] = jnp.maximum(hn, 0.0)


def _fuse_call(rows, w4, sa1, sa2, Wf, gamma, beta):
    M = sa1.shape[0]
    return pl.pallas_call(
        _fuse_body,
        out_shape=jax.ShapeDtypeStruct((M, 128), jnp.float32),
    )(rows, w4, sa1, sa2, Wf, gamma.reshape(1, 128), beta.reshape(1, 128))


def kernel(voxel_centers, spatial_features, W1a, W1b, W2a, W2b, Wf, gamma,
           beta):
    B, N, _ = voxel_centers.shape
    K = _NKP
    xt = jnp.transpose(voxel_centers, (0, 2, 1))
    kp_t = _fps_call(xt.reshape(B, 3, N // 128, 128), xt)
    keypoints = jnp.transpose(kp_t, (0, 2, 1))

    xi = (keypoints[:, :, 0] - _PC_MIN_X) / _VOX_X / _STRIDE
    yi = (keypoints[:, :, 1] - _PC_MIN_Y) / _VOX_Y / _STRIDE
    x0 = jnp.floor(xi).astype(jnp.int32)
    x1 = x0 + 1
    y0 = jnp.floor(yi).astype(jnp.int32)
    y1 = y0 + 1
    x0 = jnp.clip(x0, 0, _W - 1)
    x1 = jnp.clip(x1, 0, _W - 1)
    y0 = jnp.clip(y0, 0, _H - 1)
    y1 = jnp.clip(y1, 0, _H - 1)
    x0f = x0.astype(jnp.float32)
    x1f = x1.astype(jnp.float32)
    y0f = y0.astype(jnp.float32)
    y1f = y1.astype(jnp.float32)
    wa = (x1f - xi) * (y1f - yi)
    wb = (x1f - xi) * (yi - y0f)
    wc = (xi - x0f) * (y1f - yi)
    wd = (xi - x0f) * (yi - y0f)
    boff = (jnp.arange(B, dtype=jnp.int32) * (_H * _W))[:, None]
    ia = boff + y0 * _W + x0
    ib = boff + y1 * _W + x0
    ic = boff + y0 * _W + x1
    idd = boff + y1 * _W + x1
    idx = jnp.stack([ia, ib, ic, idd]).reshape(-1)
    table = jnp.transpose(spatial_features, (0, 2, 3, 1)).reshape(
        B * _H * _W, 256)
    rows = _bev_gather(table, idx).reshape(4, B * K, 256)
    w4 = jnp.stack([wa, wb, wc, wd]).reshape(4, B * K, 1)

    pooled1, pooled2 = _bq_call(keypoints, xt, voxel_centers,
                                W1a, W1b, W2a, W2b)
    sa1 = pooled1.reshape(B * K, _NS1)
    sa2 = pooled2.reshape(B * K, _NS2)
    return _fuse_call(rows, w4, sa1, sa2, Wf, gamma, beta)


# speedup vs baseline: 12.6122x; 1.2593x over previous
import functools

import jax
import jax.numpy as jnp
from jax import lax
from jax.experimental import pallas as pl
from jax.experimental.pallas import tpu as pltpu
from jax.experimental.pallas import tpu_sc as plsc

_PC_MIN_X = 0.0
_PC_MIN_Y = -40.0
_VOX_X = 0.05
_VOX_Y = 0.05
_STRIDE = 8
_NKP = 2048
_R1, _NS1 = 4.0, 16
_R2, _NS2 = 8.0, 32
_H, _W = 200, 176

_SC_NC, _SC_NS = 2, 16
_SC_NW = _SC_NC * _SC_NS


def _fps_body(x_ref, xs_ref, o_ref):
    B = x_ref.shape[0]
    li = (lax.broadcasted_iota(jnp.int32, (64, 128), 0) * 128
          + lax.broadcasted_iota(jnp.int32, (64, 128), 1))
    big_i = jnp.int32(2 ** 30)
    P = [(x_ref[b, 0], x_ref[b, 1], x_ref[b, 2]) for b in range(B)]

    for b in range(B):
        o_ref[b, 0, 0] = xs_ref[b, 0, 0]
        o_ref[b, 1, 0] = xs_ref[b, 1, 0]
        o_ref[b, 2, 0] = xs_ref[b, 2, 0]

    def body(i, carry):
        minds, lasts = carry
        new_minds = []
        new_lasts = []
        for b in range(B):
            X, Y, Z = P[b]
            lx, ly, lz = lasts[b]
            d = (X - lx) ** 2 + (Y - ly) ** 2 + (Z - lz) ** 2
            mind = jnp.minimum(minds[b], d)
            m = jnp.max(mind)
            cand = jnp.where(mind == m, li, big_i)
            j = jnp.min(cand)
            nx = xs_ref[b, 0, j]
            ny = xs_ref[b, 1, j]
            nz = xs_ref[b, 2, j]
            o_ref[b, 0, i] = nx
            o_ref[b, 1, i] = ny
            o_ref[b, 2, i] = nz
            new_minds.append(mind)
            new_lasts.append((nx, ny, nz))
        return (tuple(new_minds), tuple(new_lasts))

    mind0 = jnp.full((64, 128), 1e10, jnp.float32)
    lasts0 = tuple((xs_ref[b, 0, 0], xs_ref[b, 1, 0], xs_ref[b, 2, 0])
                   for b in range(B))
    lax.fori_loop(1, _NKP, body, (tuple(mind0 for _ in range(B)), lasts0))


def _fps_call(x4, xt):
    B = x4.shape[0]
    return pl.pallas_call(
        _fps_body,
        in_specs=[
            pl.BlockSpec(memory_space=pltpu.VMEM),
            pl.BlockSpec(memory_space=pltpu.SMEM),
        ],
        out_specs=pl.BlockSpec(memory_space=pltpu.SMEM),
        out_shape=jax.ShapeDtypeStruct((B, 3, _NKP), jnp.float32),
    )(x4, xt)


def _bev_gather(table, idx):
    n = idx.shape[0]
    per_w = n // _SC_NW
    chunks = per_w // 128
    mesh = plsc.VectorSubcoreMesh(core_axis_name="c", subcore_axis_name="s")

    @functools.partial(
        pl.kernel, mesh=mesh,
        out_type=jax.ShapeDtypeStruct((n, 256), jnp.float32),
        scratch_types=[
            pltpu.VMEM((128,), jnp.int32),
            pltpu.VMEM((128, 256), jnp.float32),
            pltpu.SemaphoreType.DMA,
        ],
    )
    def k(table_hbm, idx_hbm, out_hbm, idx_v, rows_v, sem):
        wid = lax.axis_index("s") * _SC_NC + lax.axis_index("c")
        for c in range(chunks):
            base = wid * per_w + c * 128
            pltpu.sync_copy(idx_hbm.at[pl.ds(base, 128)], idx_v)
            pltpu.async_copy(table_hbm.at[idx_v], rows_v, sem).wait()
            pltpu.sync_copy(rows_v, out_hbm.at[pl.ds(base, 128)])

    return k(table, idx)


_QB = 256


def _bq_body(kp_ref, xt_ref, pts_ref, w1a_ref, w1b_ref, w2a_ref, w2b_ref,
             o1_ref, o2_ref, scr_ref):
    Q = kp_ref[0]
    Xt = xt_ref[0]
    P = pts_ref[0]
    n = Xt.shape[1]
    qn = jnp.sum(Q * Q, axis=1, keepdims=True)
    xn = jnp.sum(Xt * Xt, axis=0, keepdims=True)
    qx = lax.dot_general(Q, Xt, (((1,), (0,)), ((), ())),
                         preferred_element_type=jnp.float32)
    scr_ref[...] = jnp.maximum(qn + xn - 2.0 * qx, 0.0)
    big_i = jnp.int32(2 ** 30)

    W1a = w1a_ref[...]
    W1b = w1b_ref[...]
    W2a = w2a_ref[...]
    W2b = w2b_ref[...]

    def step(s, carry):
        pooled1, pooled2 = carry
        cur = scr_ref[...]
        m = jnp.min(cur, axis=1, keepdims=True)
        li = lax.broadcasted_iota(jnp.int32, (_QB, n), 1)
        cand = jnp.where(cur == m, li, big_i)
        j = jnp.min(cand, axis=1, keepdims=True)
        ohf = (cand == j).astype(jnp.float32)
        sel = lax.dot_general(ohf, P,
                              (((1,), (0,)), ((), ())),
                              preferred_element_type=jnp.float32)
        g = sel - Q
        h1 = jnp.maximum(jnp.dot(g, W1a), 0.0)
        h1 = jnp.maximum(jnp.dot(h1, W1b), 0.0)
        h2 = jnp.maximum(jnp.dot(g, W2a), 0.0)
        h2 = jnp.maximum(jnp.dot(h2, W2b), 0.0)
        v1 = (m <= _R1 * _R1) & (s < _NS1)
        pooled1 = jnp.where(v1, jnp.maximum(pooled1, h1), pooled1)
        v2 = m <= _R2 * _R2
        pooled2 = jnp.where(v2, jnp.maximum(pooled2, h2), pooled2)
        scr_ref[...] = cur + ohf * 1e10
        return (pooled1, pooled2)

    pooled1, pooled2 = lax.fori_loop(
        0, _NS2, step,
        (jnp.zeros((_QB, _NS1), jnp.float32),
         jnp.zeros((_QB, _NS2), jnp.float32)))
    o1_ref[0] = pooled1
    o2_ref[0] = pooled2


def _bq_call(keypoints, xt, pts, W1a, W1b, W2a, W2b):
    B, N, _ = pts.shape
    grid = (B, _NKP // _QB)
    return pl.pallas_call(
        _bq_body,
        grid=grid,
        in_specs=[
            pl.BlockSpec((1, _QB, 3), lambda b, q: (b, q, 0)),
            pl.BlockSpec((1, 3, N), lambda b, q: (b, 0, 0)),
            pl.BlockSpec((1, N, 3), lambda b, q: (b, 0, 0)),
            pl.BlockSpec((3, _NS1), lambda b, q: (0, 0)),
            pl.BlockSpec((_NS1, _NS1), lambda b, q: (0, 0)),
            pl.BlockSpec((3, _NS2), lambda b, q: (0, 0)),
            pl.BlockSpec((_NS2, _NS2), lambda b, q: (0, 0)),
        ],
        out_specs=[
            pl.BlockSpec((1, _QB, _NS1), lambda b, q: (b, q, 0)),
            pl.BlockSpec((1, _QB, _NS2), lambda b, q: (b, q, 0)),
        ],
        out_shape=[
            jax.ShapeDtypeStruct((B, _NKP, _NS1), jnp.float32),
            jax.ShapeDtypeStruct((B, _NKP, _NS2), jnp.float32),
        ],
        scratch_shapes=[pltpu.VMEM((_QB, N), jnp.float32)],
    )(keypoints, xt, pts, W1a, W1b, W2a, W2b)


def _fuse_body(rows_ref, w_ref, sa1_ref, sa2_ref, wf_ref, g_ref, b_ref, o_ref):
    bev = rows_ref[0] * w_ref[0]
    for c in range(1, 4):
        bev = bev + rows_ref[c] * w_ref[c]
    feats = jnp.concatenate([bev, sa1_ref[...], sa2_ref[...]], axis=1)
    h = jnp.dot(feats, wf_ref[...], preferred_element_type=jnp.float32)
    mean = jnp.mean(h, axis=0, keepdims=True)
    var = jnp.mean((h - mean) ** 2, axis=0, keepdims=True)
    hn = (h - mean) / jnp.sqrt(var + 1e-5) * g_ref[...] + b_ref[...]
    o_ref[...] = jnp.maximum(hn, 0.0)


def _fuse_call(rows, w4, sa1, sa2, Wf, gamma, beta):
    M = sa1.shape[0]
    return pl.pallas_call(
        _fuse_body,
        out_shape=jax.ShapeDtypeStruct((M, 128), jnp.float32),
    )(rows, w4, sa1, sa2, Wf, gamma.reshape(1, 128), beta.reshape(1, 128))


def kernel(voxel_centers, spatial_features, W1a, W1b, W2a, W2b, Wf, gamma,
           beta):
    B, N, _ = voxel_centers.shape
    K = _NKP
    xt = jnp.transpose(voxel_centers, (0, 2, 1))
    kp_t = _fps_call(xt.reshape(B, 3, N // 128, 128), xt)
    keypoints = jnp.transpose(kp_t, (0, 2, 1))

    xi = (keypoints[:, :, 0] - _PC_MIN_X) / _VOX_X / _STRIDE
    yi = (keypoints[:, :, 1] - _PC_MIN_Y) / _VOX_Y / _STRIDE
    x0 = jnp.floor(xi).astype(jnp.int32)
    x1 = x0 + 1
    y0 = jnp.floor(yi).astype(jnp.int32)
    y1 = y0 + 1
    x0 = jnp.clip(x0, 0, _W - 1)
    x1 = jnp.clip(x1, 0, _W - 1)
    y0 = jnp.clip(y0, 0, _H - 1)
    y1 = jnp.clip(y1, 0, _H - 1)
    x0f = x0.astype(jnp.float32)
    x1f = x1.astype(jnp.float32)
    y0f = y0.astype(jnp.float32)
    y1f = y1.astype(jnp.float32)
    wa = (x1f - xi) * (y1f - yi)
    wb = (x1f - xi) * (yi - y0f)
    wc = (xi - x0f) * (y1f - yi)
    wd = (xi - x0f) * (yi - y0f)
    boff = (jnp.arange(B, dtype=jnp.int32) * (_H * _W))[:, None]
    ia = boff + y0 * _W + x0
    ib = boff + y1 * _W + x0
    ic = boff + y0 * _W + x1
    idd = boff + y1 * _W + x1
    idx = jnp.stack([ia, ib, ic, idd]).reshape(-1)
    table = jnp.transpose(spatial_features, (0, 2, 3, 1)).reshape(
        B * _H * _W, 256)
    rows = _bev_gather(table, idx).reshape(4, B * K, 256)
    w4 = jnp.stack([wa, wb, wc, wd]).reshape(4, B * K, 1)

    pooled1, pooled2 = _bq_call(keypoints, xt, voxel_centers,
                                W1a, W1b, W2a, W2b)
    sa1 = pooled1.reshape(B * K, _NS1)
    sa2 = pooled2.reshape(B * K, _NS2)
    return _fuse_call(rows, w4, sa1, sa2, Wf, gamma, beta)


# speedup vs baseline: 14.9967x; 1.1891x over previous
import functools

import jax
import jax.numpy as jnp
from jax import lax
from jax.experimental import pallas as pl
from jax.experimental.pallas import tpu as pltpu
from jax.experimental.pallas import tpu_sc as plsc

_PC_MIN_X = 0.0
_PC_MIN_Y = -40.0
_VOX_X = 0.05
_VOX_Y = 0.05
_STRIDE = 8
_NKP = 2048
_R1, _NS1 = 4.0, 16
_R2, _NS2 = 8.0, 32
_H, _W = 200, 176

_SC_NC, _SC_NS = 2, 16
_SC_NW = _SC_NC * _SC_NS


def _fps_body(x_ref, xs_ref, o_ref):
    B = x_ref.shape[0]
    li = (lax.broadcasted_iota(jnp.int32, (64, 128), 0) * 128
          + lax.broadcasted_iota(jnp.int32, (64, 128), 1)).astype(jnp.float32)
    big_f = jnp.float32(3e10)
    P = [(x_ref[b, 0], x_ref[b, 1], x_ref[b, 2]) for b in range(B)]

    for b in range(B):
        o_ref[b, 0, 0] = xs_ref[b, 0, 0]
        o_ref[b, 1, 0] = xs_ref[b, 1, 0]
        o_ref[b, 2, 0] = xs_ref[b, 2, 0]

    def body(i, carry):
        minds, lasts = carry
        new_minds = []
        new_lasts = []
        for b in range(B):
            X, Y, Z = P[b]
            lx, ly, lz = lasts[b]
            d = (X - lx) ** 2 + (Y - ly) ** 2 + (Z - lz) ** 2
            mind = jnp.minimum(minds[b], d)
            m = jnp.max(mind)
            cand = jnp.where(mind == m, li, big_f)
            j = jnp.min(cand).astype(jnp.int32)
            nx = xs_ref[b, 0, j]
            ny = xs_ref[b, 1, j]
            nz = xs_ref[b, 2, j]
            o_ref[b, 0, i] = nx
            o_ref[b, 1, i] = ny
            o_ref[b, 2, i] = nz
            new_minds.append(mind)
            new_lasts.append((nx, ny, nz))
        return (tuple(new_minds), tuple(new_lasts))

    mind0 = jnp.full((64, 128), 1e10, jnp.float32)
    lasts0 = tuple((xs_ref[b, 0, 0], xs_ref[b, 1, 0], xs_ref[b, 2, 0])
                   for b in range(B))
    lax.fori_loop(1, _NKP, body, (tuple(mind0 for _ in range(B)), lasts0))


def _fps_call(x4, xt):
    B = x4.shape[0]
    return pl.pallas_call(
        _fps_body,
        in_specs=[
            pl.BlockSpec(memory_space=pltpu.VMEM),
            pl.BlockSpec(memory_space=pltpu.SMEM),
        ],
        out_specs=pl.BlockSpec(memory_space=pltpu.SMEM),
        out_shape=jax.ShapeDtypeStruct((B, 3, _NKP), jnp.float32),
    )(x4, xt)


def _bev_gather(table, idx):
    n = idx.shape[0]
    per_w = n // _SC_NW
    chunks = per_w // 128
    mesh = plsc.VectorSubcoreMesh(core_axis_name="c", subcore_axis_name="s")

    @functools.partial(
        pl.kernel, mesh=mesh,
        out_type=jax.ShapeDtypeStruct((n, 256), jnp.float32),
        scratch_types=[
            pltpu.VMEM((128,), jnp.int32),
            pltpu.VMEM((128, 256), jnp.float32),
            pltpu.SemaphoreType.DMA,
        ],
    )
    def k(table_hbm, idx_hbm, out_hbm, idx_v, rows_v, sem):
        wid = lax.axis_index("s") * _SC_NC + lax.axis_index("c")
        for c in range(chunks):
            base = wid * per_w + c * 128
            pltpu.sync_copy(idx_hbm.at[pl.ds(base, 128)], idx_v)
            pltpu.async_copy(table_hbm.at[idx_v], rows_v, sem).wait()
            pltpu.sync_copy(rows_v, out_hbm.at[pl.ds(base, 128)])

    return k(table, idx)


_QB = 256


def _bq_body(kp_ref, xt_ref, pts_ref, w1a_ref, w1b_ref, w2a_ref, w2b_ref,
             o1_ref, o2_ref, scr_ref):
    Q = kp_ref[0]
    Xt = xt_ref[0]
    P = pts_ref[0]
    n = Xt.shape[1]
    qn = jnp.sum(Q * Q, axis=1, keepdims=True)
    xn = jnp.sum(Xt * Xt, axis=0, keepdims=True)
    qx = lax.dot_general(Q, Xt, (((1,), (0,)), ((), ())),
                         preferred_element_type=jnp.float32)
    scr_ref[...] = jnp.maximum(qn + xn - 2.0 * qx, 0.0)
    big_f = jnp.float32(3e10)

    W1a = w1a_ref[...]
    W1b = w1b_ref[...]
    W2a = w2a_ref[...]
    W2b = w2b_ref[...]

    def step(s, carry):
        pooled1, pooled2 = carry
        cur = scr_ref[...]
        m = jnp.min(cur, axis=1, keepdims=True)
        li = lax.broadcasted_iota(jnp.int32, (_QB, n), 1).astype(jnp.float32)
        cand = jnp.where(cur == m, li, big_f)
        j = jnp.min(cand, axis=1, keepdims=True)
        ohf = (cand == j).astype(jnp.float32)
        sel = lax.dot_general(ohf, P,
                              (((1,), (0,)), ((), ())),
                              preferred_element_type=jnp.float32)
        g = sel - Q
        h1 = jnp.maximum(jnp.dot(g, W1a), 0.0)
        h1 = jnp.maximum(jnp.dot(h1, W1b), 0.0)
        h2 = jnp.maximum(jnp.dot(g, W2a), 0.0)
        h2 = jnp.maximum(jnp.dot(h2, W2b), 0.0)
        v1 = (m <= _R1 * _R1) & (s < _NS1)
        pooled1 = jnp.where(v1, jnp.maximum(pooled1, h1), pooled1)
        v2 = m <= _R2 * _R2
        pooled2 = jnp.where(v2, jnp.maximum(pooled2, h2), pooled2)
        scr_ref[...] = cur + ohf * 1e10
        return (pooled1, pooled2)

    pooled1, pooled2 = lax.fori_loop(
        0, _NS2, step,
        (jnp.zeros((_QB, _NS1), jnp.float32),
         jnp.zeros((_QB, _NS2), jnp.float32)))
    o1_ref[0] = pooled1
    o2_ref[0] = pooled2


def _bq_call(keypoints, xt, pts, W1a, W1b, W2a, W2b):
    B, N, _ = pts.shape
    grid = (B, _NKP // _QB)
    return pl.pallas_call(
        _bq_body,
        grid=grid,
        in_specs=[
            pl.BlockSpec((1, _QB, 3), lambda b, q: (b, q, 0)),
            pl.BlockSpec((1, 3, N), lambda b, q: (b, 0, 0)),
            pl.BlockSpec((1, N, 3), lambda b, q: (b, 0, 0)),
            pl.BlockSpec((3, _NS1), lambda b, q: (0, 0)),
            pl.BlockSpec((_NS1, _NS1), lambda b, q: (0, 0)),
            pl.BlockSpec((3, _NS2), lambda b, q: (0, 0)),
            pl.BlockSpec((_NS2, _NS2), lambda b, q: (0, 0)),
        ],
        out_specs=[
            pl.BlockSpec((1, _QB, _NS1), lambda b, q: (b, q, 0)),
            pl.BlockSpec((1, _QB, _NS2), lambda b, q: (b, q, 0)),
        ],
        out_shape=[
            jax.ShapeDtypeStruct((B, _NKP, _NS1), jnp.float32),
            jax.ShapeDtypeStruct((B, _NKP, _NS2), jnp.float32),
        ],
        scratch_shapes=[pltpu.VMEM((_QB, N), jnp.float32)],
    )(keypoints, xt, pts, W1a, W1b, W2a, W2b)


def _fuse_body(rows_ref, w_ref, sa1_ref, sa2_ref, wf_ref, g_ref, b_ref, o_ref):
    bev = rows_ref[0] * w_ref[0]
    for c in range(1, 4):
        bev = bev + rows_ref[c] * w_ref[c]
    feats = jnp.concatenate([bev, sa1_ref[...], sa2_ref[...]], axis=1)
    h = jnp.dot(feats, wf_ref[...], preferred_element_type=jnp.float32)
    mean = jnp.mean(h, axis=0, keepdims=True)
    var = jnp.mean((h - mean) ** 2, axis=0, keepdims=True)
    hn = (h - mean) / jnp.sqrt(var + 1e-5) * g_ref[...] + b_ref[...]
    o_ref[...] = jnp.maximum(hn, 0.0)


def _fuse_call(rows, w4, sa1, sa2, Wf, gamma, beta):
    M = sa1.shape[0]
    return pl.pallas_call(
        _fuse_body,
        out_shape=jax.ShapeDtypeStruct((M, 128), jnp.float32),
    )(rows, w4, sa1, sa2, Wf, gamma.reshape(1, 128), beta.reshape(1, 128))


def kernel(voxel_centers, spatial_features, W1a, W1b, W2a, W2b, Wf, gamma,
           beta):
    B, N, _ = voxel_centers.shape
    K = _NKP
    xt = jnp.transpose(voxel_centers, (0, 2, 1))
    kp_t = _fps_call(xt.reshape(B, 3, N // 128, 128), xt)
    keypoints = jnp.transpose(kp_t, (0, 2, 1))

    xi = (keypoints[:, :, 0] - _PC_MIN_X) / _VOX_X / _STRIDE
    yi = (keypoints[:, :, 1] - _PC_MIN_Y) / _VOX_Y / _STRIDE
    x0 = jnp.floor(xi).astype(jnp.int32)
    x1 = x0 + 1
    y0 = jnp.floor(yi).astype(jnp.int32)
    y1 = y0 + 1
    x0 = jnp.clip(x0, 0, _W - 1)
    x1 = jnp.clip(x1, 0, _W - 1)
    y0 = jnp.clip(y0, 0, _H - 1)
    y1 = jnp.clip(y1, 0, _H - 1)
    x0f = x0.astype(jnp.float32)
    x1f = x1.astype(jnp.float32)
    y0f = y0.astype(jnp.float32)
    y1f = y1.astype(jnp.float32)
    wa = (x1f - xi) * (y1f - yi)
    wb = (x1f - xi) * (yi - y0f)
    wc = (xi - x0f) * (y1f - yi)
    wd = (xi - x0f) * (yi - y0f)
    boff = (jnp.arange(B, dtype=jnp.int32) * (_H * _W))[:, None]
    ia = boff + y0 * _W + x0
    ib = boff + y1 * _W + x0
    ic = boff + y0 * _W + x1
    idd = boff + y1 * _W + x1
    idx = jnp.stack([ia, ib, ic, idd]).reshape(-1)
    table = jnp.transpose(spatial_features, (0, 2, 3, 1)).reshape(
        B * _H * _W, 256)
    rows = _bev_gather(table, idx).reshape(4, B * K, 256)
    w4 = jnp.stack([wa, wb, wc, wd]).reshape(4, B * K, 1)

    pooled1, pooled2 = _bq_call(keypoints, xt, voxel_centers,
                                W1a, W1b, W2a, W2b)
    sa1 = pooled1.reshape(B * K, _NS1)
    sa2 = pooled2.reshape(B * K, _NS2)
    return _fuse_call(rows, w4, sa1, sa2, Wf, gamma, beta)
